# Initial kernel scaffold; baseline (speedup 1.0000x reference)
import jax
import jax.numpy as jnp
from jax.experimental import pallas as pl


def kernel(pos, positional_embeddings):
    raise NotImplementedError("write your pallas kernel here")


# speedup vs baseline: 2.5930x; 2.5930x over previous
import functools

import jax
import jax.numpy as jnp
from jax import lax
from jax.experimental import pallas as pl
from jax.experimental.pallas import tpu as pltpu
from jax.experimental.pallas import tpu_sc as plsc

HIDDEN = 768
WIDTH = 32
NUM_POS = WIDTH * WIDTH + 1
N = 262144

NC, NS, L = 2, 16, 16
NW = NC * NS
BPW = N // NW
CHUNK = 64
NCHUNK = BPW // CHUNK

_mesh = plsc.VectorSubcoreMesh(core_axis_name="c", subcore_axis_name="s")


@functools.partial(
    pl.kernel,
    out_type=jax.ShapeDtypeStruct((N, HIDDEN), jnp.float32),
    mesh=_mesh,
    scratch_types=[
        pltpu.VMEM((BPW,), jnp.float32),
        pltpu.VMEM((BPW,), jnp.float32),
        pltpu.VMEM((BPW,), jnp.int32),
        pltpu.VMEM((2, CHUNK, HIDDEN), jnp.float32),
        pltpu.SemaphoreType.DMA,
        pltpu.SemaphoreType.DMA,
    ],
)
def _uv_pos_gather(x_hbm, y_hbm, table_hbm, out_hbm, x_v, y_v, idx_v, rows_v, g0, g1):
    wid = lax.axis_index("s") * NC + lax.axis_index("c")
    base = wid * BPW

    pltpu.sync_copy(x_hbm.at[pl.ds(base, BPW)], x_v)
    pltpu.sync_copy(y_hbm.at[pl.ds(base, BPW)], y_v)

    def idx_body(j, carry):
        x = x_v[pl.ds(L * j, L)]
        y = y_v[pl.ds(L * j, L)]
        idx = (x * WIDTH).astype(jnp.int32) * WIDTH + (y * WIDTH).astype(jnp.int32) + 1
        idx_v[pl.ds(L * j, L)] = idx
        return carry

    lax.fori_loop(0, BPW // L, idx_body, 0)

    gsems = (g0, g1)

    def start_gather(c, slot):
        return pltpu.async_copy(
            table_hbm.at[idx_v.at[pl.ds(c * CHUNK, CHUNK)]],
            rows_v.at[slot],
            gsems[slot],
        )

    start_gather(0, 0)
    start_gather(1, 1)

    def gather_body(t, carry):
        for b in range(2):
            c = 2 * t + b
            pltpu.make_async_copy(
                table_hbm.at[idx_v.at[pl.ds(c * CHUNK, CHUNK)]],
                rows_v.at[b],
                gsems[b],
            ).wait()
            pltpu.sync_copy(rows_v.at[b], out_hbm.at[pl.ds(base + c * CHUNK, CHUNK)])

            @pl.when(c + 2 < NCHUNK)
            def _():
                start_gather(c + 2, b)

        return carry

    lax.fori_loop(0, NCHUNK // 2, gather_body, 0)


def kernel(pos, positional_embeddings):
    table = positional_embeddings.reshape(NUM_POS, HIDDEN)
    out = _uv_pos_gather(pos[:, 0], pos[:, 1], table)
    return out[None]


# speedup vs baseline: 2.5953x; 1.0009x over previous
import functools

import jax
import jax.numpy as jnp
from jax import lax
from jax.experimental import pallas as pl
from jax.experimental.pallas import tpu as pltpu
from jax.experimental.pallas import tpu_sc as plsc

HIDDEN = 768
WIDTH = 32
NUM_POS = WIDTH * WIDTH + 1
N = 262144

NC, NS, L = 2, 16, 16
NW = NC * NS
BPW = N // NW
NBUF = 4
CHUNK = 32
NCHUNK = BPW // CHUNK

_mesh = plsc.VectorSubcoreMesh(core_axis_name="c", subcore_axis_name="s")


@functools.partial(
    pl.kernel,
    out_type=jax.ShapeDtypeStruct((N, HIDDEN), jnp.float32),
    mesh=_mesh,
    scratch_types=[
        pltpu.VMEM((BPW,), jnp.float32),
        pltpu.VMEM((BPW,), jnp.float32),
        pltpu.VMEM((BPW,), jnp.int32),
        pltpu.VMEM((NBUF, CHUNK, HIDDEN), jnp.float32),
    ] + [pltpu.SemaphoreType.DMA] * (2 * NBUF),
)
def _uv_pos_gather(x_hbm, y_hbm, table_hbm, out_hbm, x_v, y_v, idx_v, rows_v,
                   g0, g1, g2, g3, w0, w1, w2, w3):
    gsems = (g0, g1, g2, g3)
    wsems = (w0, w1, w2, w3)
    wid = lax.axis_index("s") * NC + lax.axis_index("c")
    base = wid * BPW

    pltpu.sync_copy(x_hbm.at[pl.ds(base, BPW)], x_v)
    pltpu.sync_copy(y_hbm.at[pl.ds(base, BPW)], y_v)

    def idx_body(j, carry):
        x = x_v[pl.ds(L * j, L)]
        y = y_v[pl.ds(L * j, L)]
        idx = (x * WIDTH).astype(jnp.int32) * WIDTH + (y * WIDTH).astype(jnp.int32) + 1
        idx_v[pl.ds(L * j, L)] = idx
        return carry

    lax.fori_loop(0, BPW // L, idx_body, 0)

    def start_gather(c, b):
        pltpu.async_copy(
            table_hbm.at[idx_v.at[pl.ds(c * CHUNK, CHUNK)]],
            rows_v.at[b],
            gsems[b],
        )

    def out_copy(c, b):
        return pltpu.make_async_copy(
            rows_v.at[b],
            out_hbm.at[pl.ds(base + c * CHUNK, CHUNK)],
            wsems[b],
        )

    start_gather(0, 0)
    start_gather(1, 1)

    def gather_body(t, carry):
        for b in range(NBUF):
            c = NBUF * t + b
            bd = (b + 2) % NBUF
            pltpu.make_async_copy(
                table_hbm.at[idx_v.at[pl.ds(c * CHUNK, CHUNK)]],
                rows_v.at[b],
                gsems[b],
            ).wait()
            out_copy(c, b).start()

            @pl.when(c >= 2)
            def _():
                out_copy(c - 2, bd).wait()

            @pl.when(c + 2 < NCHUNK)
            def _():
                start_gather(c + 2, bd)

        return carry

    lax.fori_loop(0, NCHUNK // NBUF, gather_body, 0)

    out_copy(NCHUNK - 2, (NCHUNK - 2) % NBUF).wait()
    out_copy(NCHUNK - 1, (NCHUNK - 1) % NBUF).wait()


def kernel(pos, positional_embeddings):
    table = positional_embeddings.reshape(NUM_POS, HIDDEN)
    out = _uv_pos_gather(pos[:, 0], pos[:, 1], table)
    return out[None]
